# baseline (speedup 1.0000x reference)
import functools

import jax
import jax.numpy as jnp
from jax import lax
from jax.experimental import pallas as pl
from jax.experimental.pallas import tpu as pltpu
from jax.experimental.pallas import tpu_sc as plsc

NC = 2
NS = 16
NW = NC * NS
LANES = 16


def _loss2_body(pred_ref, binned_ref, ws_ref, out_ref):
    x = pred_ref[0]
    m = jnp.max(x, axis=0)
    s = jnp.sum(jnp.exp(x - m[None]), axis=0)
    logz = m + jnp.log(s)
    t = binned_ref[0]
    qio = lax.broadcasted_iota(jnp.int32, x.shape, 0)
    wsp = jnp.sum(jnp.where(qio == t[None], ws_ref[...][:, :, None], 0.0),
                  axis=0)

    @pl.when(pl.program_id(0) == 0)
    def _():
        out_ref[...] = jnp.zeros_like(out_ref)

    out_ref[...] += jnp.sum(wsp * logz).reshape(1, 1)


def _loss2_call(pred, binned3, ws2, rh):
    B, Q, H, W = pred.shape
    n_j = H // rh
    grid = (B * n_j,)
    return pl.pallas_call(
        _loss2_body,
        grid=grid,
        in_specs=[
            pl.BlockSpec((1, Q, rh, W), lambda g: (g // n_j, 0, g % n_j, 0)),
            pl.BlockSpec((1, rh, W), lambda g: (g // n_j, g % n_j, 0)),
            pl.BlockSpec((Q, 1), lambda g: (0, 0)),
        ],
        out_specs=pl.BlockSpec((1, 1), lambda g: (0, 0)),
        out_shape=jax.ShapeDtypeStruct((1, 1), jnp.float32),
    )(pred, binned3, ws2)


def _sc1_body(Q, HW, K, chunk,
              pred_hbm, binned_hbm, knn_hbm, kw_hbm, w_hbm,
              out_hbm,
              knn_v, kw_v, w_v, t_v, addr_v, kws_v, g_v, acc_v, sem):
    cid = lax.axis_index("c")
    sid = lax.axis_index("s")
    wid = sid * NC + cid
    base = wid * chunk
    b = base // HW
    hw0 = base - b * HW
    pred_base = b * (Q * HW)

    pltpu.sync_copy(knn_hbm, knn_v)
    pltpu.sync_copy(kw_hbm, kw_v)
    pltpu.sync_copy(w_hbm, w_v)
    pltpu.sync_copy(binned_hbm.at[pl.ds(base, chunk)], t_v)

    lane = jnp.arange(LANES, dtype=jnp.int32)
    n_grp = chunk // LANES
    n_tab = knn_v.shape[0] // LANES

    def prescale(j, carry):
        fl = j * LANES + lane
        wv = plsc.load_gather(w_v, [fl // K])
        kw_v[pl.ds(j * LANES, LANES)] = kw_v[pl.ds(j * LANES, LANES)] * wv
        knn_v[pl.ds(j * LANES, LANES)] = knn_v[pl.ds(j * LANES, LANES)] * HW
        return carry

    lax.fori_loop(0, n_tab, prescale, 0)
    hwbase = pred_base + hw0 + lane

    def phase1_one(i):
        t16 = t_v[pl.ds(i * LANES, LANES)]
        tk = t16 * K
        hw = hwbase + i * LANES
        for k in range(K):
            ck = plsc.load_gather(knn_v, [tk + k])
            kwk = plsc.load_gather(kw_v, [tk + k])
            g = i * K + k
            addr_v[pl.ds(g * LANES, LANES)] = ck + hw
            kws_v[pl.ds(g * LANES, LANES)] = kwk

    def phase1(i2, carry):
        phase1_one(2 * i2)
        phase1_one(2 * i2 + 1)
        return carry

    def phase2_one(i):
        acc = jnp.zeros((LANES,), jnp.float32)
        for k in range(K):
            g = i * K + k
            acc = acc + (kws_v[pl.ds(g * LANES, LANES)]
                         * g_v[pl.ds(g * LANES, LANES)])
        return acc

    def phase2(i2, carry):
        acc_v[...] = (acc_v[...] + phase2_one(2 * i2)) + phase2_one(2 * i2 + 1)
        return carry

    NB = 4
    gpb = n_grp // NB
    hpb = gpb // 2
    epb = gpb * LANES * K
    acc_v[...] = jnp.zeros((LANES,), jnp.float32)
    descs = []
    for j in range(NB):
        lax.fori_loop(j * hpb, (j + 1) * hpb, phase1, 0)
        descs.append(pltpu.async_copy(
            pred_hbm.at[addr_v.at[pl.ds(j * epb, epb)]],
            g_v.at[pl.ds(j * epb, epb)], sem))
        if j >= 1:
            descs[j - 1].wait()
            lax.fori_loop((j - 1) * hpb, j * hpb, phase2, 0)
    descs[NB - 1].wait()
    lax.fori_loop((NB - 1) * hpb, NB * hpb, phase2, 0)
    pltpu.sync_copy(acc_v, out_hbm.at[wid])


def _sc1_call(pred_flat, binned_flat, knn_flat, kw_flat, w_pad,
              Q, HW, K, chunk):
    N = binned_flat.shape[0]
    mesh = plsc.VectorSubcoreMesh(core_axis_name="c", subcore_axis_name="s")
    body = functools.partial(_sc1_body, Q, HW, K, chunk)
    return pl.kernel(
        body,
        out_type=jax.ShapeDtypeStruct((NW, LANES), jnp.float32),
        mesh=mesh,
        compiler_params=pltpu.CompilerParams(needs_layout_passes=False),
        scratch_types=[
            pltpu.VMEM((knn_flat.shape[0],), jnp.int32),
            pltpu.VMEM((kw_flat.shape[0],), jnp.float32),
            pltpu.VMEM((w_pad.shape[0],), jnp.float32),
            pltpu.VMEM((chunk,), jnp.int32),
            pltpu.VMEM((chunk * K,), jnp.int32),
            pltpu.VMEM((chunk * K,), jnp.float32),
            pltpu.VMEM((chunk * K,), jnp.float32),
            pltpu.VMEM((LANES,), jnp.float32),
            pltpu.SemaphoreType.DMA,
        ],
    )(pred_flat, binned_flat, knn_flat, kw_flat, w_pad)


def kernel(pred, _color, binned_color, knn_idx, knn_weights, weights):
    B, Q, H, W = pred.shape
    K = knn_idx.shape[1]
    HW = H * W
    N = B * HW
    chunk = N // NW

    pred_flat = pred.reshape(-1)
    binned_flat = binned_color.reshape(-1).astype(jnp.int32)
    binned3 = binned_color.reshape(B, H, W).astype(jnp.int32)
    knn_flat = jnp.pad(knn_idx.astype(jnp.int32).reshape(-1), (0, -(Q * K) % 8))
    kw_flat = jnp.pad(knn_weights.astype(jnp.float32).reshape(-1),
                      (0, -(Q * K) % 8))
    w_pad = jnp.pad(weights.astype(jnp.float32), (0, -Q % 8))
    ws2 = (weights.astype(jnp.float32)
           * jnp.sum(knn_weights.astype(jnp.float32), axis=1)).reshape(Q, 1)

    part1 = _sc1_call(pred_flat, binned_flat, knn_flat, kw_flat,
                      w_pad, Q, HW, K, chunk)
    part2 = _loss2_call(pred, binned3, ws2, rh=64)
    return -(jnp.sum(part1) - part2[0, 0]) / N


# speedup vs baseline: 1.0232x; 1.0232x over previous
import functools

import jax
import jax.numpy as jnp
from jax import lax
from jax.experimental import pallas as pl
from jax.experimental.pallas import tpu as pltpu
from jax.experimental.pallas import tpu_sc as plsc

NC = 2
NS = 16
NW = NC * NS
LANES = 16


def _loss2_body(pred_ref, binned_ref, ws_ref, out_ref):
    x = pred_ref[0]
    m = jnp.max(x, axis=0)
    s = jnp.sum(jnp.exp(x - m[None]), axis=0)
    logz = m + jnp.log(s)
    t = binned_ref[0]
    qio = lax.broadcasted_iota(jnp.int32, x.shape, 0)
    wsp = jnp.sum(jnp.where(qio == t[None], ws_ref[...][:, :, None], 0.0),
                  axis=0)

    @pl.when(pl.program_id(0) == 0)
    def _():
        out_ref[...] = jnp.zeros_like(out_ref)

    out_ref[...] += jnp.sum(wsp * logz).reshape(1, 1)


def _loss2_call(pred, binned3, ws2, rh):
    B, Q, H, W = pred.shape
    n_j = H // rh
    grid = (B * n_j,)
    return pl.pallas_call(
        _loss2_body,
        grid=grid,
        compiler_params=pltpu.CompilerParams(
            vmem_limit_bytes=100 * 1024 * 1024),
        in_specs=[
            pl.BlockSpec((1, Q, rh, W), lambda g: (g // n_j, 0, g % n_j, 0)),
            pl.BlockSpec((1, rh, W), lambda g: (g // n_j, g % n_j, 0)),
            pl.BlockSpec((Q, 1), lambda g: (0, 0)),
        ],
        out_specs=pl.BlockSpec((1, 1), lambda g: (0, 0)),
        out_shape=jax.ShapeDtypeStruct((1, 1), jnp.float32),
    )(pred, binned3, ws2)


def _sc1_body(Q, HW, K, chunk,
              pred_hbm, binned_hbm, knn_hbm, kw_hbm, w_hbm,
              out_hbm,
              knn_v, kw_v, w_v, t_v, addr_v, kws_v, g_v, acc_v, sem):
    cid = lax.axis_index("c")
    sid = lax.axis_index("s")
    wid = sid * NC + cid
    base = wid * chunk
    b = base // HW
    hw0 = base - b * HW
    pred_base = b * (Q * HW)

    pltpu.sync_copy(knn_hbm, knn_v)
    pltpu.sync_copy(kw_hbm, kw_v)
    pltpu.sync_copy(w_hbm, w_v)
    pltpu.sync_copy(binned_hbm.at[pl.ds(base, chunk)], t_v)

    lane = jnp.arange(LANES, dtype=jnp.int32)
    n_grp = chunk // LANES
    n_tab = knn_v.shape[0] // LANES

    def prescale(j, carry):
        fl = j * LANES + lane
        wv = plsc.load_gather(w_v, [fl // K])
        kw_v[pl.ds(j * LANES, LANES)] = kw_v[pl.ds(j * LANES, LANES)] * wv
        knn_v[pl.ds(j * LANES, LANES)] = knn_v[pl.ds(j * LANES, LANES)] * HW
        return carry

    lax.fori_loop(0, n_tab, prescale, 0)
    hwbase = pred_base + hw0 + lane

    def phase1_one(i):
        t16 = t_v[pl.ds(i * LANES, LANES)]
        tk = t16 * K
        hw = hwbase + i * LANES
        for k in range(K):
            ck = plsc.load_gather(knn_v, [tk + k])
            kwk = plsc.load_gather(kw_v, [tk + k])
            g = i * K + k
            addr_v[pl.ds(g * LANES, LANES)] = ck + hw
            kws_v[pl.ds(g * LANES, LANES)] = kwk

    def phase1(i2, carry):
        phase1_one(2 * i2)
        phase1_one(2 * i2 + 1)
        return carry

    def phase2_one(i):
        acc = jnp.zeros((LANES,), jnp.float32)
        for k in range(K):
            g = i * K + k
            acc = acc + (kws_v[pl.ds(g * LANES, LANES)]
                         * g_v[pl.ds(g * LANES, LANES)])
        return acc

    def phase2(i2, carry):
        acc_v[...] = (acc_v[...] + phase2_one(2 * i2)) + phase2_one(2 * i2 + 1)
        return carry

    NB = 4
    gpb = n_grp // NB
    hpb = gpb // 2
    epb = gpb * LANES * K
    acc_v[...] = jnp.zeros((LANES,), jnp.float32)
    descs = []
    for j in range(NB):
        lax.fori_loop(j * hpb, (j + 1) * hpb, phase1, 0)
        descs.append(pltpu.async_copy(
            pred_hbm.at[addr_v.at[pl.ds(j * epb, epb)]],
            g_v.at[pl.ds(j * epb, epb)], sem))
        if j >= 1:
            descs[j - 1].wait()
            lax.fori_loop((j - 1) * hpb, j * hpb, phase2, 0)
    descs[NB - 1].wait()
    lax.fori_loop((NB - 1) * hpb, NB * hpb, phase2, 0)
    pltpu.sync_copy(acc_v, out_hbm.at[wid])


def _sc1_call(pred_flat, binned_flat, knn_flat, kw_flat, w_pad,
              Q, HW, K, chunk):
    N = binned_flat.shape[0]
    mesh = plsc.VectorSubcoreMesh(core_axis_name="c", subcore_axis_name="s")
    body = functools.partial(_sc1_body, Q, HW, K, chunk)
    return pl.kernel(
        body,
        out_type=jax.ShapeDtypeStruct((NW, LANES), jnp.float32),
        mesh=mesh,
        compiler_params=pltpu.CompilerParams(needs_layout_passes=False),
        scratch_types=[
            pltpu.VMEM((knn_flat.shape[0],), jnp.int32),
            pltpu.VMEM((kw_flat.shape[0],), jnp.float32),
            pltpu.VMEM((w_pad.shape[0],), jnp.float32),
            pltpu.VMEM((chunk,), jnp.int32),
            pltpu.VMEM((chunk * K,), jnp.int32),
            pltpu.VMEM((chunk * K,), jnp.float32),
            pltpu.VMEM((chunk * K,), jnp.float32),
            pltpu.VMEM((LANES,), jnp.float32),
            pltpu.SemaphoreType.DMA,
        ],
    )(pred_flat, binned_flat, knn_flat, kw_flat, w_pad)


def kernel(pred, _color, binned_color, knn_idx, knn_weights, weights):
    B, Q, H, W = pred.shape
    K = knn_idx.shape[1]
    HW = H * W
    N = B * HW
    chunk = N // NW

    pred_flat = pred.reshape(-1)
    binned_flat = binned_color.reshape(-1).astype(jnp.int32)
    binned3 = binned_color.reshape(B, H, W).astype(jnp.int32)
    knn_flat = jnp.pad(knn_idx.astype(jnp.int32).reshape(-1), (0, -(Q * K) % 8))
    kw_flat = jnp.pad(knn_weights.astype(jnp.float32).reshape(-1),
                      (0, -(Q * K) % 8))
    w_pad = jnp.pad(weights.astype(jnp.float32), (0, -Q % 8))
    ws2 = (weights.astype(jnp.float32)
           * jnp.sum(knn_weights.astype(jnp.float32), axis=1)).reshape(Q, 1)

    part1 = _sc1_call(pred_flat, binned_flat, knn_flat, kw_flat,
                      w_pad, Q, HW, K, chunk)
    part2 = _loss2_call(pred, binned3, ws2, rh=128)
    return -(jnp.sum(part1) - part2[0, 0]) / N


# speedup vs baseline: 1.1009x; 1.0759x over previous
import functools

import jax
import jax.numpy as jnp
from jax import lax
from jax.experimental import pallas as pl
from jax.experimental.pallas import tpu as pltpu
from jax.experimental.pallas import tpu_sc as plsc

NC = 2
NS = 16
NW = NC * NS
LANES = 16


def _logz_body(pred_ref, out_ref):
    x = pred_ref[0]
    m = jnp.max(x, axis=0)
    s = jnp.sum(jnp.exp(x - m[None]), axis=0)
    out_ref[0] = m + jnp.log(s)


def _logz_call(pred, rh):
    B, Q, H, W = pred.shape
    n_j = H // rh
    grid = (B * n_j,)
    return pl.pallas_call(
        _logz_body,
        grid=grid,
        in_specs=[pl.BlockSpec((1, Q, rh, W),
                               lambda g: (g // n_j, 0, g % n_j, 0))],
        out_specs=pl.BlockSpec((1, rh, W), lambda g: (g // n_j, g % n_j, 0)),
        out_shape=jax.ShapeDtypeStruct((B, H, W), jnp.float32),
    )(pred)


def _sc1_body(Q, HW, K, chunk,
              pred_hbm, binned_hbm, knn_hbm, kw_hbm, w_hbm,
              out_hbm, ws_hbm,
              knn_v, kw_v, w_v, t_v, addr_v, kws_v, g_v, ws_v, acc_v, sem):
    cid = lax.axis_index("c")
    sid = lax.axis_index("s")
    wid = sid * NC + cid
    base = wid * chunk
    b = base // HW
    hw0 = base - b * HW
    pred_base = b * (Q * HW)

    pltpu.sync_copy(knn_hbm, knn_v)
    pltpu.sync_copy(kw_hbm, kw_v)
    pltpu.sync_copy(w_hbm, w_v)
    pltpu.sync_copy(binned_hbm.at[pl.ds(base, chunk)], t_v)

    lane = jnp.arange(LANES, dtype=jnp.int32)
    n_grp = chunk // LANES
    n_tab = knn_v.shape[0] // LANES

    def prescale(j, carry):
        fl = j * LANES + lane
        wv = plsc.load_gather(w_v, [jnp.minimum(fl // K, Q - 1)])
        kw_v[pl.ds(j * LANES, LANES)] = kw_v[pl.ds(j * LANES, LANES)] * wv
        knn_v[pl.ds(j * LANES, LANES)] = knn_v[pl.ds(j * LANES, LANES)] * HW
        return carry

    lax.fori_loop(0, n_tab, prescale, 0)
    hwbase = pred_base + hw0 + lane

    def phase1_one(i):
        t16 = t_v[pl.ds(i * LANES, LANES)]
        tk = t16 * K
        hw = hwbase + i * LANES
        ws = jnp.zeros((LANES,), jnp.float32)
        for k in range(K):
            ck = plsc.load_gather(knn_v, [tk + k])
            kwk = plsc.load_gather(kw_v, [tk + k])
            g = i * K + k
            addr_v[pl.ds(g * LANES, LANES)] = ck + hw
            kws_v[pl.ds(g * LANES, LANES)] = kwk
            ws = ws + kwk
        ws_v[pl.ds(i * LANES, LANES)] = ws

    def phase1(i2, carry):
        phase1_one(2 * i2)
        phase1_one(2 * i2 + 1)
        return carry

    def phase2_one(i):
        acc = jnp.zeros((LANES,), jnp.float32)
        for k in range(K):
            g = i * K + k
            acc = acc + (kws_v[pl.ds(g * LANES, LANES)]
                         * g_v[pl.ds(g * LANES, LANES)])
        return acc

    def phase2(i2, carry):
        acc_v[...] = (acc_v[...] + phase2_one(2 * i2)) + phase2_one(2 * i2 + 1)
        return carry

    NB = 4
    gpb = n_grp // NB
    hpb = gpb // 2
    epb = gpb * LANES * K
    acc_v[...] = jnp.zeros((LANES,), jnp.float32)
    descs = []
    for j in range(NB):
        lax.fori_loop(j * hpb, (j + 1) * hpb, phase1, 0)
        descs.append(pltpu.async_copy(
            pred_hbm.at[addr_v.at[pl.ds(j * epb, epb)]],
            g_v.at[pl.ds(j * epb, epb)], sem))
        if j >= 1:
            descs[j - 1].wait()
            lax.fori_loop((j - 1) * hpb, j * hpb, phase2, 0)
    pltpu.sync_copy(ws_v, ws_hbm.at[pl.ds(base, chunk)])
    descs[NB - 1].wait()
    lax.fori_loop((NB - 1) * hpb, NB * hpb, phase2, 0)
    pltpu.sync_copy(acc_v, out_hbm.at[wid])


def _sc1_call(pred_flat, binned_flat, knn_flat, kw_flat, w_pad,
              Q, HW, K, chunk):
    N = binned_flat.shape[0]
    mesh = plsc.VectorSubcoreMesh(core_axis_name="c", subcore_axis_name="s")
    body = functools.partial(_sc1_body, Q, HW, K, chunk)
    return pl.kernel(
        body,
        out_type=(jax.ShapeDtypeStruct((NW, LANES), jnp.float32),
                  jax.ShapeDtypeStruct((N,), jnp.float32)),
        mesh=mesh,
        compiler_params=pltpu.CompilerParams(needs_layout_passes=False),
        scratch_types=[
            pltpu.VMEM(((Q * K + LANES - 1) // LANES * LANES,), jnp.int32),
            pltpu.VMEM(((Q * K + LANES - 1) // LANES * LANES,), jnp.float32),
            pltpu.VMEM((Q,), jnp.float32),
            pltpu.VMEM((chunk,), jnp.int32),
            pltpu.VMEM((chunk * K,), jnp.int32),
            pltpu.VMEM((chunk * K,), jnp.float32),
            pltpu.VMEM((chunk * K,), jnp.float32),
            pltpu.VMEM((chunk,), jnp.float32),
            pltpu.VMEM((LANES,), jnp.float32),
            pltpu.SemaphoreType.DMA,
        ],
    )(pred_flat, binned_flat, knn_flat, kw_flat, w_pad)


def _dot_body(a_ref, b_ref, out_ref):
    out_ref[...] = jnp.sum(a_ref[...] * b_ref[...]).reshape(1, 1)


def _dot_call(a, b):
    return pl.pallas_call(
        _dot_body,
        out_shape=jax.ShapeDtypeStruct((1, 1), jnp.float32),
    )(a, b)


def kernel(pred, _color, binned_color, knn_idx, knn_weights, weights):
    B, Q, H, W = pred.shape
    K = knn_idx.shape[1]
    HW = H * W
    N = B * HW
    chunk = N // NW

    pred_flat = pred.reshape(-1)
    binned_flat = binned_color.reshape(-1).astype(jnp.int32)
    knn_flat = jnp.pad(knn_idx.astype(jnp.int32).reshape(-1), (0, -(Q * K) % 16))
    kw_flat = jnp.pad(knn_weights.astype(jnp.float32).reshape(-1),
                      (0, -(Q * K) % 16))
    w_pad = weights.astype(jnp.float32)

    part1, ws_flat = _sc1_call(pred_flat, binned_flat, knn_flat, kw_flat,
                               w_pad, Q, HW, K, chunk)
    logz = _logz_call(pred, rh=128).reshape(-1)
    part2 = _dot_call(ws_flat, logz)
    return -(jnp.sum(part1) - part2[0, 0]) / N
